# Initial kernel scaffold; baseline (speedup 1.0000x reference)
import jax
import jax.numpy as jnp
from jax.experimental import pallas as pl


def kernel(x, edge_index, W, b):
    raise NotImplementedError("write your pallas kernel here")


# speedup vs baseline: 8.7611x; 8.7611x over previous
import functools

import jax
import jax.numpy as jnp
from jax import lax
from jax.experimental import pallas as pl
from jax.experimental.pallas import tpu as pltpu
from jax.experimental.pallas import tpu_sc as plsc

D = 128
NC = 2
NS = 16
L = 16
CH = 2048


def _mesh():
    return plsc.VectorSubcoreMesh(core_axis_name="c", subcore_axis_name="s")


def _params():
    return pltpu.CompilerParams(needs_layout_passes=False)


def _route_body(np_, ech, cap, safe_src, dst_hi,
                src2, dst2, degp, glh, dlh, cnth,
                src_c, dst_c, gl, dl, hist, cnt_v):
    c = lax.axis_index("c")
    s = lax.axis_index("s")
    w = s * NC + c
    rw = np_ // NS
    base = s * rw

    ones = jnp.ones((L,), jnp.float32)
    z = jnp.zeros((L,), jnp.float32)
    safe = jnp.full((L,), safe_src, jnp.int32)
    zi = jnp.zeros((L,), jnp.int32)

    def pre(i, carry):
        gl[pl.ds(i * L, L)] = safe
        dl[pl.ds(i * L, L)] = zi
        return carry

    lax.fori_loop(0, cap // L, pre, 0)

    def zh(i, carry):
        hist[pl.ds(i * L, L)] = z
        return carry

    lax.fori_loop(0, rw // L, zh, 0)

    def chunk(chi, pos):
        pltpu.sync_copy(src2.at[c, chi], src_c)
        pltpu.sync_copy(dst2.at[c, chi], dst_c)

        def vec(i, p):
            sv = src_c[pl.ds(i * L, L)]
            dv = dst_c[pl.ds(i * L, L)]
            m = jnp.logical_and(dv >= base, dv < base + rw)
            loc = jnp.where(m, dv - base, 0)
            plsc.store_compressed(gl.at[pl.ds(p, L)], sv, mask=m)
            plsc.store_compressed(dl.at[pl.ds(p, L)], loc, mask=m)
            plsc.addupdate_scatter(hist, [loc], ones, mask=m)
            cnt = plsc.all_reduce_population_count(m)
            return p + cnt[0]

        return lax.fori_loop(0, CH // L, vec, pos)

    pos = lax.fori_loop(0, ech, chunk, jnp.int32(0))

    pltpu.sync_copy(hist, degp.at[c, pl.ds(base, rw)])
    pltpu.sync_copy(gl, glh.at[w])
    pltpu.sync_copy(dl, dlh.at[w])
    cnt_v[:] = jnp.full((L,), pos, jnp.int32)
    pltpu.sync_copy(cnt_v, cnth.at[w])


def _make_route_call(np_, ech, cap, safe_src, dst_hi):
    nw = NC * NS
    return pl.kernel(
        functools.partial(_route_body, np_, ech, cap, safe_src, dst_hi),
        out_type=(
            jax.ShapeDtypeStruct((NC, np_), jnp.float32),
            jax.ShapeDtypeStruct((nw, cap), jnp.int32),
            jax.ShapeDtypeStruct((nw, cap), jnp.int32),
            jax.ShapeDtypeStruct((nw, L), jnp.int32),
        ),
        mesh=_mesh(),
        compiler_params=_params(),
        scratch_types=[
            pltpu.VMEM((CH,), jnp.int32),
            pltpu.VMEM((CH,), jnp.int32),
            pltpu.VMEM((cap,), jnp.int32),
            pltpu.VMEM((cap,), jnp.int32),
            pltpu.VMEM((np_ // NS,), jnp.float32),
            pltpu.VMEM((L,), jnp.int32),
        ],
    )


def _acc_body(np_, cap, y_hbm, gl2, dl2, cnth, accp,
              gl_v, dl_v, rows, acc, cnt_v, sem):
    c = lax.axis_index("c")
    s = lax.axis_index("s")
    w = s * NC + c
    rw = np_ // NS
    base = s * rw

    pltpu.sync_copy(gl2.at[w], gl_v)
    pltpu.sync_copy(dl2.at[w], dl_v)
    pltpu.sync_copy(cnth.at[w], cnt_v)

    z = jnp.zeros((L,), jnp.float32)

    def za(r, carry):
        for j in range(D // L):
            acc[r, pl.ds(j * L, L)] = z
        return carry

    lax.fori_loop(0, rw, za, 0)

    cnt = cnt_v[pl.ds(0, L)][0]
    nq = (cnt + 127) >> 7

    def body(q, carry):
        pltpu.async_copy(y_hbm.at[gl_v.at[q]], rows, sem).wait()

        def grp(r, carry2):
            dlv = dl_v[q, pl.ds(r * L, L)]
            for i16 in range(L):
                d = dlv[i16]
                ri = r * L + i16
                for j in range(D // L):
                    plsc.addupdate(acc.at[d, pl.ds(j * L, L)],
                                   rows[ri, pl.ds(j * L, L)])
            return carry2

        return lax.fori_loop(0, 128 // L, grp, carry)

    lax.fori_loop(0, nq, body, 0)

    pltpu.sync_copy(acc, accp.at[c, pl.ds(base, rw)])


def _make_acc_call(np_, cap):
    nw = NC * NS
    return pl.kernel(
        functools.partial(_acc_body, np_, cap),
        out_type=jax.ShapeDtypeStruct((NC, np_, D), jnp.float32),
        mesh=_mesh(),
        compiler_params=_params(),
        scratch_types=[
            pltpu.VMEM((cap // 128, 128), jnp.int32),
            pltpu.VMEM((cap // 128, 128), jnp.int32),
            pltpu.VMEM((128, D), jnp.float32),
            pltpu.VMEM((np_ // NS, D), jnp.float32),
            pltpu.VMEM((L,), jnp.int32),
            pltpu.SemaphoreType.DMA,
        ],
    )


def _prep_body(x_ref, wt_ref, d0_ref, d1_ref, y_ref, dis_ref):
    deg = d0_ref[...] + d1_ref[...] + 1.0
    dis = lax.rsqrt(deg)
    xw = jnp.dot(x_ref[...], wt_ref[...], preferred_element_type=jnp.float32)
    y_ref[...] = dis * xw
    dis_ref[...] = dis


def _make_prep_call(np_, bn):
    return pl.pallas_call(
        _prep_body,
        grid=(np_ // bn,),
        in_specs=[
            pl.BlockSpec((bn, D), lambda i: (i, 0)),
            pl.BlockSpec((D, D), lambda i: (0, 0)),
            pl.BlockSpec((bn, 1), lambda i: (i, 0)),
            pl.BlockSpec((bn, 1), lambda i: (i, 0)),
        ],
        out_specs=[
            pl.BlockSpec((bn, D), lambda i: (i, 0)),
            pl.BlockSpec((bn, 1), lambda i: (i, 0)),
        ],
        out_shape=[
            jax.ShapeDtypeStruct((np_, D), jnp.float32),
            jax.ShapeDtypeStruct((np_, 1), jnp.float32),
        ],
    )


def _fin_body(a0_ref, a1_ref, y_ref, dis_ref, b_ref, o_ref):
    o_ref[...] = dis_ref[...] * (a0_ref[...] + a1_ref[...] + y_ref[...]) + b_ref[...]


def _make_fin_call(np_, bn):
    return pl.pallas_call(
        _fin_body,
        grid=(np_ // bn,),
        in_specs=[
            pl.BlockSpec((bn, D), lambda i: (i, 0)),
            pl.BlockSpec((bn, D), lambda i: (i, 0)),
            pl.BlockSpec((bn, D), lambda i: (i, 0)),
            pl.BlockSpec((bn, 1), lambda i: (i, 0)),
            pl.BlockSpec((1, D), lambda i: (0, 0)),
        ],
        out_specs=pl.BlockSpec((bn, D), lambda i: (i, 0)),
        out_shape=jax.ShapeDtypeStruct((np_, D), jnp.float32),
    )


def kernel(x, edge_index, W, b):
    n = x.shape[0]
    e = edge_index.shape[1]

    np_ = ((n + 1 + NS * 128 - 1) // (NS * 128)) * (NS * 128)
    ep = ((e + NC * CH - 1) // (NC * CH)) * (NC * CH)
    ech = ep // NC // CH
    cap = ((int(ep // NC // NS * 1.27) + 127) // 128) * 128

    src = edge_index[0].astype(jnp.int32)
    dst = edge_index[1].astype(jnp.int32)
    pad_e = ep - e
    src = jnp.concatenate([src, jnp.full((pad_e,), n, jnp.int32)])
    dst = jnp.concatenate([dst, jnp.full((pad_e,), np_ - 1, jnp.int32)])
    src2 = src.reshape(NC, ech, CH)
    dst2 = dst.reshape(NC, ech, CH)

    x_pad = jnp.pad(x, ((0, np_ - n), (0, 0)))
    wt = W.T

    degp, glh, dlh, cnth = _make_route_call(np_, ech, cap, n, np_ - 1)(
        src2, dst2)
    d0 = degp[0].reshape(np_, 1)
    d1 = degp[1].reshape(np_, 1)
    y, dis = _make_prep_call(np_, 1024)(x_pad, wt, d0, d1)
    gl2 = glh.reshape(NC * NS, cap // 128, 128)
    dl2 = dlh.reshape(NC * NS, cap // 128, 128)
    accp = _make_acc_call(np_, cap)(y, gl2, dl2, cnth)
    out_full = _make_fin_call(np_, 1024)(accp[0], accp[1], y, dis,
                                         b.reshape(1, D))
    return out_full[:n]


# speedup vs baseline: 10.8272x; 1.2358x over previous
import functools

import jax
import jax.numpy as jnp
from jax import lax
from jax.experimental import pallas as pl
from jax.experimental.pallas import tpu as pltpu
from jax.experimental.pallas import tpu_sc as plsc

D = 128
NC = 2
NS = 16
L = 16
CH = 2048


def _mesh():
    return plsc.VectorSubcoreMesh(core_axis_name="c", subcore_axis_name="s")


def _params():
    return pltpu.CompilerParams(needs_layout_passes=False)


def _route_body(np_, ech, cap, safe_src, dst_hi,
                src2, dst2, degp, glh, dlh, cnth,
                src_c, dst_c, gl, dl, hist, cnt_v):
    c = lax.axis_index("c")
    s = lax.axis_index("s")
    w = s * NC + c
    rw = np_ // NS
    base = s * rw

    ones = jnp.ones((L,), jnp.float32)
    z = jnp.zeros((L,), jnp.float32)
    safe = jnp.full((L,), safe_src, jnp.int32)
    zi = jnp.zeros((L,), jnp.int32)

    def pre(i, carry):
        gl[pl.ds(i * L, L)] = safe
        dl[pl.ds(i * L, L)] = zi
        return carry

    lax.fori_loop(0, cap // L, pre, 0)

    def zh(i, carry):
        hist[pl.ds(i * L, L)] = z
        return carry

    lax.fori_loop(0, rw // L, zh, 0)

    def chunk(chi, pos):
        pltpu.sync_copy(src2.at[c, chi], src_c)
        pltpu.sync_copy(dst2.at[c, chi], dst_c)

        def vec(i, p):
            sv = src_c[pl.ds(i * L, L)]
            dv = dst_c[pl.ds(i * L, L)]
            m = jnp.logical_and(dv >= base, dv < base + rw)
            loc = jnp.where(m, dv - base, 0)
            plsc.store_compressed(gl.at[pl.ds(p, L)], sv, mask=m)
            plsc.store_compressed(dl.at[pl.ds(p, L)], loc, mask=m)
            plsc.addupdate_scatter(hist, [loc], ones, mask=m)
            cnt = plsc.all_reduce_population_count(m)
            return p + cnt[0]

        return lax.fori_loop(0, CH // L, vec, pos)

    pos = lax.fori_loop(0, ech, chunk, jnp.int32(0))

    pltpu.sync_copy(hist, degp.at[c, pl.ds(base, rw)])
    pltpu.sync_copy(gl, glh.at[w])
    pltpu.sync_copy(dl, dlh.at[w])
    cnt_v[:] = jnp.full((L,), pos, jnp.int32)
    pltpu.sync_copy(cnt_v, cnth.at[w])


def _make_route_call(np_, ech, cap, safe_src, dst_hi):
    nw = NC * NS
    return pl.kernel(
        functools.partial(_route_body, np_, ech, cap, safe_src, dst_hi),
        out_type=(
            jax.ShapeDtypeStruct((NC, np_), jnp.float32),
            jax.ShapeDtypeStruct((nw, cap), jnp.int32),
            jax.ShapeDtypeStruct((nw, cap), jnp.int32),
            jax.ShapeDtypeStruct((nw, L), jnp.int32),
        ),
        mesh=_mesh(),
        compiler_params=_params(),
        scratch_types=[
            pltpu.VMEM((CH,), jnp.int32),
            pltpu.VMEM((CH,), jnp.int32),
            pltpu.VMEM((cap,), jnp.int32),
            pltpu.VMEM((cap,), jnp.int32),
            pltpu.VMEM((np_ // NS,), jnp.float32),
            pltpu.VMEM((L,), jnp.int32),
        ],
    )


def _acc_body(np_, cap, y_hbm, gl2, dl2, cnth, accp,
              gl_v, dl_v, rows, acc, cnt_v, sem):
    c = lax.axis_index("c")
    s = lax.axis_index("s")
    w = s * NC + c
    rw = np_ // NS
    base = s * rw

    pltpu.sync_copy(gl2.at[w], gl_v)
    pltpu.sync_copy(dl2.at[w], dl_v)
    pltpu.sync_copy(cnth.at[w], cnt_v)

    z = jnp.zeros((L,), jnp.float32)

    def za(r, carry):
        for j in range(D // L):
            acc[r, pl.ds(j * L, L)] = z
        return carry

    lax.fori_loop(0, rw, za, 0)

    cnt = cnt_v[pl.ds(0, L)][0]
    nq = (cnt + 127) >> 7

    def body(q, carry):
        pltpu.async_copy(y_hbm.at[gl_v.at[q]], rows, sem).wait()

        def grp(r, carry2):
            dlv = dl_v[q, pl.ds(r * L, L)]
            ds_ = [dlv[i16] for i16 in range(L)]
            for i16 in range(L):
                ri = r * L + i16
                segs = [rows[ri, pl.ds(j * L, L)] for j in range(D // L)]
                for j in range(D // L):
                    plsc.addupdate(acc.at[ds_[i16], pl.ds(j * L, L)], segs[j])
            return carry2

        return lax.fori_loop(0, 128 // L, grp, carry)

    lax.fori_loop(0, nq, body, 0)

    pltpu.sync_copy(acc, accp.at[c, pl.ds(base, rw)])


def _make_acc_call(np_, cap):
    nw = NC * NS
    return pl.kernel(
        functools.partial(_acc_body, np_, cap),
        out_type=jax.ShapeDtypeStruct((NC, np_, D), jnp.float32),
        mesh=_mesh(),
        compiler_params=_params(),
        scratch_types=[
            pltpu.VMEM((cap // 128, 128), jnp.int32),
            pltpu.VMEM((cap // 128, 128), jnp.int32),
            pltpu.VMEM((128, D), jnp.float32),
            pltpu.VMEM((np_ // NS, D), jnp.float32),
            pltpu.VMEM((L,), jnp.int32),
            pltpu.SemaphoreType.DMA,
        ],
    )


def _prep_body(x_ref, wt_ref, d0_ref, d1_ref, y_ref, dis_ref):
    deg = d0_ref[...] + d1_ref[...] + 1.0
    dis = lax.rsqrt(deg)
    xw = jnp.dot(x_ref[...], wt_ref[...], preferred_element_type=jnp.float32)
    y_ref[...] = dis * xw
    dis_ref[...] = dis


def _make_prep_call(np_, bn):
    return pl.pallas_call(
        _prep_body,
        grid=(np_ // bn,),
        in_specs=[
            pl.BlockSpec((bn, D), lambda i: (i, 0)),
            pl.BlockSpec((D, D), lambda i: (0, 0)),
            pl.BlockSpec((bn, 1), lambda i: (i, 0)),
            pl.BlockSpec((bn, 1), lambda i: (i, 0)),
        ],
        out_specs=[
            pl.BlockSpec((bn, D), lambda i: (i, 0)),
            pl.BlockSpec((bn, 1), lambda i: (i, 0)),
        ],
        out_shape=[
            jax.ShapeDtypeStruct((np_, D), jnp.float32),
            jax.ShapeDtypeStruct((np_, 1), jnp.float32),
        ],
    )


def _fin_body(a0_ref, a1_ref, y_ref, dis_ref, b_ref, o_ref):
    o_ref[...] = dis_ref[...] * (a0_ref[...] + a1_ref[...] + y_ref[...]) + b_ref[...]


def _make_fin_call(np_, bn):
    return pl.pallas_call(
        _fin_body,
        grid=(np_ // bn,),
        in_specs=[
            pl.BlockSpec((bn, D), lambda i: (i, 0)),
            pl.BlockSpec((bn, D), lambda i: (i, 0)),
            pl.BlockSpec((bn, D), lambda i: (i, 0)),
            pl.BlockSpec((bn, 1), lambda i: (i, 0)),
            pl.BlockSpec((1, D), lambda i: (0, 0)),
        ],
        out_specs=pl.BlockSpec((bn, D), lambda i: (i, 0)),
        out_shape=jax.ShapeDtypeStruct((np_, D), jnp.float32),
    )


def kernel(x, edge_index, W, b):
    n = x.shape[0]
    e = edge_index.shape[1]

    np_ = ((n + 1 + NS * 128 - 1) // (NS * 128)) * (NS * 128)
    ep = ((e + NC * CH - 1) // (NC * CH)) * (NC * CH)
    ech = ep // NC // CH
    cap = ((int(ep // NC // NS * 1.27) + 127) // 128) * 128

    src = edge_index[0].astype(jnp.int32)
    dst = edge_index[1].astype(jnp.int32)
    pad_e = ep - e
    src = jnp.concatenate([src, jnp.full((pad_e,), n, jnp.int32)])
    dst = jnp.concatenate([dst, jnp.full((pad_e,), np_ - 1, jnp.int32)])
    src2 = src.reshape(NC, ech, CH)
    dst2 = dst.reshape(NC, ech, CH)

    x_pad = jnp.pad(x, ((0, np_ - n), (0, 0)))
    wt = W.T

    degp, glh, dlh, cnth = _make_route_call(np_, ech, cap, n, np_ - 1)(
        src2, dst2)
    d0 = degp[0].reshape(np_, 1)
    d1 = degp[1].reshape(np_, 1)
    y, dis = _make_prep_call(np_, 1024)(x_pad, wt, d0, d1)
    gl2 = glh.reshape(NC * NS, cap // 128, 128)
    dl2 = dlh.reshape(NC * NS, cap // 128, 128)
    accp = _make_acc_call(np_, cap)(y, gl2, dl2, cnth)
    out_full = _make_fin_call(np_, 1024)(accp[0], accp[1], y, dis,
                                         b.reshape(1, D))
    return out_full[:n]


# speedup vs baseline: 11.3259x; 1.0461x over previous
import functools

import jax
import jax.numpy as jnp
from jax import lax
from jax.experimental import pallas as pl
from jax.experimental.pallas import tpu as pltpu
from jax.experimental.pallas import tpu_sc as plsc

D = 128
NC = 2
NS = 16
L = 16
CH = 2048


def _mesh():
    return plsc.VectorSubcoreMesh(core_axis_name="c", subcore_axis_name="s")


def _params():
    return pltpu.CompilerParams(needs_layout_passes=False)


def _route_body(np_, ech, cap, safe_src, dst_hi,
                src2, dst2, degp, glh, dlh, cnth,
                src_c, dst_c, gl, dl, hist, cnt_v):
    c = lax.axis_index("c")
    s = lax.axis_index("s")
    w = s * NC + c
    rw = np_ // NS
    base = s * rw

    ones = jnp.ones((L,), jnp.float32)
    z = jnp.zeros((L,), jnp.float32)
    safe = jnp.full((L,), safe_src, jnp.int32)
    zi = jnp.zeros((L,), jnp.int32)

    def pre(i, carry):
        gl[pl.ds(i * L, L)] = safe
        dl[pl.ds(i * L, L)] = zi
        return carry

    lax.fori_loop(0, cap // L, pre, 0)

    def zh(i, carry):
        hist[pl.ds(i * L, L)] = z
        return carry

    lax.fori_loop(0, rw // L, zh, 0)

    hcap = cap // 2

    def chunk(chi, ps):
        pltpu.sync_copy(src2.at[c, chi], src_c)
        pltpu.sync_copy(dst2.at[c, chi], dst_c)

        def vec(i, ps2):
            p0, p1 = ps2
            o = i * 2 * L
            sv0 = src_c[pl.ds(o, L)]
            dv0 = dst_c[pl.ds(o, L)]
            sv1 = src_c[pl.ds(o + L, L)]
            dv1 = dst_c[pl.ds(o + L, L)]
            m0 = jnp.logical_and(dv0 >= base, dv0 < base + rw)
            m1 = jnp.logical_and(dv1 >= base, dv1 < base + rw)
            l0 = jnp.where(m0, dv0 - base, 0)
            l1 = jnp.where(m1, dv1 - base, 0)
            plsc.store_compressed(gl.at[pl.ds(p0, L)], sv0, mask=m0)
            plsc.store_compressed(dl.at[pl.ds(p0, L)], l0, mask=m0)
            plsc.addupdate_scatter(hist, [l0], ones, mask=m0)
            plsc.store_compressed(gl.at[pl.ds(hcap + p1, L)], sv1, mask=m1)
            plsc.store_compressed(dl.at[pl.ds(hcap + p1, L)], l1, mask=m1)
            plsc.addupdate_scatter(hist, [l1], ones, mask=m1)
            c0 = plsc.all_reduce_population_count(m0)
            c1 = plsc.all_reduce_population_count(m1)
            return (p0 + c0[0], p1 + c1[0])

        return lax.fori_loop(0, CH // (2 * L), vec, ps)

    p0, p1 = lax.fori_loop(0, ech, chunk, (jnp.int32(0), jnp.int32(0)))

    pltpu.sync_copy(hist, degp.at[c, pl.ds(base, rw)])
    pltpu.sync_copy(gl, glh.at[w])
    pltpu.sync_copy(dl, dlh.at[w])
    lane = lax.iota(jnp.int32, L)
    cnt_v[:] = jnp.where(lane == 0, p0, p1)
    pltpu.sync_copy(cnt_v, cnth.at[w])


def _make_route_call(np_, ech, cap, safe_src, dst_hi):
    nw = NC * NS
    return pl.kernel(
        functools.partial(_route_body, np_, ech, cap, safe_src, dst_hi),
        out_type=(
            jax.ShapeDtypeStruct((NC, np_), jnp.float32),
            jax.ShapeDtypeStruct((nw, cap), jnp.int32),
            jax.ShapeDtypeStruct((nw, cap), jnp.int32),
            jax.ShapeDtypeStruct((nw, L), jnp.int32),
        ),
        mesh=_mesh(),
        compiler_params=_params(),
        scratch_types=[
            pltpu.VMEM((CH,), jnp.int32),
            pltpu.VMEM((CH,), jnp.int32),
            pltpu.VMEM((cap,), jnp.int32),
            pltpu.VMEM((cap,), jnp.int32),
            pltpu.VMEM((np_ // NS,), jnp.float32),
            pltpu.VMEM((L,), jnp.int32),
        ],
    )


def _acc_body(np_, cap, y_hbm, gl2, dl2, cnth, accp,
              gl_a, gl_b, dl_v, rows_a, rows_b, acc, cnt_v, sem_a, sem_b):
    c = lax.axis_index("c")
    s = lax.axis_index("s")
    w = s * NC + c
    rw = np_ // NS
    base = s * rw

    pltpu.sync_copy(dl2.at[w], dl_v)
    pltpu.sync_copy(cnth.at[w], cnt_v)

    z = jnp.zeros((L,), jnp.float32)

    def za(r, carry):
        for j in range(D // L):
            acc[r, pl.ds(j * L, L)] = z
        return carry

    lax.fori_loop(0, rw, za, 0)

    cv = cnt_v[pl.ds(0, L)]
    hrows = cap // 2 // 128

    def rmw(q, rows):
        def grp(r, carry2):
            dlv = dl_v[q, pl.ds(r * L, L)]
            ds_ = [dlv[i16] for i16 in range(L)]
            for i16 in range(L):
                ri = r * L + i16
                segs = [rows[ri, pl.ds(j * L, L)] for j in range(D // L)]
                for j in range(D // L):
                    plsc.addupdate(acc.at[ds_[i16], pl.ds(j * L, L)], segs[j])
            return carry2

        lax.fori_loop(0, 128 // L, grp, 0)

    def run(off, nq):
        @pl.when(nq > 0)
        def _():
            pltpu.sync_copy(gl2.at[w, off], gl_a)
            pltpu.async_copy(y_hbm.at[gl_a], rows_a, sem_a)

        def body(q, carry):
            for par, cur_g, cur_r, cur_s, nxt_g, nxt_r, nxt_s in (
                (0, gl_a, rows_a, sem_a, gl_b, rows_b, sem_b),
                (1, gl_b, rows_b, sem_b, gl_a, rows_a, sem_a),
            ):
                @pl.when(lax.rem(q, 2) == par)
                def _(cur_g=cur_g, cur_r=cur_r, cur_s=cur_s,
                      nxt_g=nxt_g, nxt_r=nxt_r, nxt_s=nxt_s):
                    @pl.when(q + 1 < nq)
                    def __():
                        pltpu.sync_copy(gl2.at[w, off + q + 1], nxt_g)
                        pltpu.async_copy(y_hbm.at[nxt_g], nxt_r, nxt_s)

                    pltpu.make_async_copy(y_hbm.at[cur_g], cur_r, cur_s).wait()
                    rmw(off + q, cur_r)

            return carry

        lax.fori_loop(0, nq, body, 0)

    run(0, (cv[0] + 127) >> 7)
    run(hrows, (cv[1] + 127) >> 7)

    pltpu.sync_copy(acc, accp.at[c, pl.ds(base, rw)])


def _make_acc_call(np_, cap):
    nw = NC * NS
    return pl.kernel(
        functools.partial(_acc_body, np_, cap),
        out_type=jax.ShapeDtypeStruct((NC, np_, D), jnp.float32),
        mesh=_mesh(),
        compiler_params=_params(),
        scratch_types=[
            pltpu.VMEM((128,), jnp.int32),
            pltpu.VMEM((128,), jnp.int32),
            pltpu.VMEM((cap // 128, 128), jnp.int32),
            pltpu.VMEM((128, D), jnp.float32),
            pltpu.VMEM((128, D), jnp.float32),
            pltpu.VMEM((np_ // NS, D), jnp.float32),
            pltpu.VMEM((L,), jnp.int32),
            pltpu.SemaphoreType.DMA,
            pltpu.SemaphoreType.DMA,
        ],
    )


def _prep_body(x_ref, wt_ref, d0_ref, d1_ref, y_ref, dis_ref):
    deg = d0_ref[...] + d1_ref[...] + 1.0
    dis = lax.rsqrt(deg)
    xw = jnp.dot(x_ref[...], wt_ref[...], preferred_element_type=jnp.float32)
    y_ref[...] = dis * xw
    dis_ref[...] = dis


def _make_prep_call(np_, bn):
    return pl.pallas_call(
        _prep_body,
        grid=(np_ // bn,),
        in_specs=[
            pl.BlockSpec((bn, D), lambda i: (i, 0)),
            pl.BlockSpec((D, D), lambda i: (0, 0)),
            pl.BlockSpec((bn, 1), lambda i: (i, 0)),
            pl.BlockSpec((bn, 1), lambda i: (i, 0)),
        ],
        out_specs=[
            pl.BlockSpec((bn, D), lambda i: (i, 0)),
            pl.BlockSpec((bn, 1), lambda i: (i, 0)),
        ],
        out_shape=[
            jax.ShapeDtypeStruct((np_, D), jnp.float32),
            jax.ShapeDtypeStruct((np_, 1), jnp.float32),
        ],
    )


def _fin_body(a0_ref, a1_ref, y_ref, dis_ref, b_ref, o_ref):
    o_ref[...] = dis_ref[...] * (a0_ref[...] + a1_ref[...] + y_ref[...]) + b_ref[...]


def _make_fin_call(np_, bn):
    return pl.pallas_call(
        _fin_body,
        grid=(np_ // bn,),
        in_specs=[
            pl.BlockSpec((bn, D), lambda i: (i, 0)),
            pl.BlockSpec((bn, D), lambda i: (i, 0)),
            pl.BlockSpec((bn, D), lambda i: (i, 0)),
            pl.BlockSpec((bn, 1), lambda i: (i, 0)),
            pl.BlockSpec((1, D), lambda i: (0, 0)),
        ],
        out_specs=pl.BlockSpec((bn, D), lambda i: (i, 0)),
        out_shape=jax.ShapeDtypeStruct((np_, D), jnp.float32),
    )


def kernel(x, edge_index, W, b):
    n = x.shape[0]
    e = edge_index.shape[1]

    np_ = ((n + 1 + NS * 128 - 1) // (NS * 128)) * (NS * 128)
    ep = ((e + NC * CH - 1) // (NC * CH)) * (NC * CH)
    ech = ep // NC // CH
    cap = ((int(ep // NC // NS * 1.27) + 255) // 256) * 256

    src = edge_index[0].astype(jnp.int32)
    dst = edge_index[1].astype(jnp.int32)
    pad_e = ep - e
    src = jnp.concatenate([src, jnp.full((pad_e,), n, jnp.int32)])
    dst = jnp.concatenate([dst, jnp.full((pad_e,), np_ - 1, jnp.int32)])
    src2 = src.reshape(NC, ech, CH)
    dst2 = dst.reshape(NC, ech, CH)

    x_pad = jnp.pad(x, ((0, np_ - n), (0, 0)))
    wt = W.T

    degp, glh, dlh, cnth = _make_route_call(np_, ech, cap, n, np_ - 1)(
        src2, dst2)
    d0 = degp[0].reshape(np_, 1)
    d1 = degp[1].reshape(np_, 1)
    y, dis = _make_prep_call(np_, 1024)(x_pad, wt, d0, d1)
    gl2 = glh.reshape(NC * NS, cap // 128, 128)
    dl2 = dlh.reshape(NC * NS, cap // 128, 128)
    accp = _make_acc_call(np_, cap)(y, gl2, dl2, cnth)
    out_full = _make_fin_call(np_, 1024)(accp[0], accp[1], y, dis,
                                         b.reshape(1, D))
    return out_full[:n]
